# baseline (speedup 1.0000x reference)
import dataclasses
import functools

import jax
import jax.numpy as jnp
from jax import lax
from jax.experimental import pallas as pl
from jax.experimental.pallas import tpu as pltpu
from jax.experimental.pallas import tpu_sc as plsc

R = 262144
NC = 2
NS = 16
NW = NC * NS
NPW = R // NW
C = 2048
NCHUNK = NPW // C
L = 16
G = C // L

BP = 1024

_LN2 = 0.6931471805599453
_SQRT2 = 1.4142135623730951


def _prep_body(kmix_ref, kmpos_ref, rpos_ref, stoich_ref, ixsub_ref,
               kiix_ref, ixki_ref, lkcat_ref, lenz_ref,
               sub0_ref, sub1_ref, ekm0_ref, ekm1_ref, s0_ref, s1_ref,
               kiix1_ref, ixki1_ref, kelog_ref):
    kmix = kmix_ref[...]
    p = kmpos_ref[...]
    q = rpos_ref[...]
    st = stoich_ref[...]
    ixs = ixsub_ref[...]
    km0, km1 = kmix[:, 0], kmix[:, 1]
    st0, st1 = st[:, 0], st[:, 1]
    ekm0_ref[...] = jnp.where(p[:, 0] == 0, km0, km1)
    ekm1_ref[...] = jnp.where(p[:, 1] == 0, km0, km1)
    s0_ref[...] = jnp.abs(jnp.where(q[:, 0] == 0, st0, st1))
    s1_ref[...] = jnp.abs(jnp.where(q[:, 1] == 0, st0, st1))
    sub0_ref[...] = ixs[:, 0]
    sub1_ref[...] = ixs[:, 1]
    kiix1_ref[...] = kiix_ref[...][:, 0]
    ixki1_ref[...] = ixki_ref[...][:, 0]
    kelog_ref[...] = lkcat_ref[...] + lenz_ref[...]


def _prep(km_ix, kmpos, rpos, stoich, ixsub, kiix, ixki, lkcat, lenz):
    n2 = pl.BlockSpec((BP, 2), lambda i: (i, 0))
    n1c = pl.BlockSpec((BP, 1), lambda i: (i, 0))
    n1 = pl.BlockSpec((BP,), lambda i: (i,))
    oi = jax.ShapeDtypeStruct((R,), jnp.int32)
    of = jax.ShapeDtypeStruct((R,), jnp.float32)
    return pl.pallas_call(
        _prep_body,
        grid=(R // BP,),
        in_specs=[n2, n2, n2, n2, n2, n1c, n1c, n1, n1],
        out_specs=[n1] * 9,
        out_shape=[oi, oi, oi, oi, of, of, oi, oi, of],
    )(km_ix, kmpos, rpos, stoich, ixsub, kiix, ixki, lkcat, lenz)


def _ln1p_pos(r):
    x = r + 1.0
    xi = lax.bitcast_convert_type(x, jnp.int32)
    e = lax.shift_right_logical(xi, 23) - 127
    m = lax.bitcast_convert_type(
        jnp.bitwise_or(jnp.bitwise_and(xi, 0x007FFFFF), 0x3F800000),
        jnp.float32)
    big = m > _SQRT2
    m = jnp.where(big, m * 0.5, m)
    e = e + jnp.where(big, 1, 0)
    u = (m - 1.0) / (m + 1.0)
    u2 = u * u
    pp = u * (2.0 + u2 * (0.6666666666666666
                          + u2 * (0.4 + u2 * 0.2857142857142857)))
    return e.astype(jnp.float32) * _LN2 + pp


def _mm_body(conc_h, lkm_h, lki_h, kelog_h, sub0_h, sub1_h, ekm0_h, ekm1_h,
             s0_h, s1_h, kiix_h, ixki_h, out_h,
             b_sub0, b_sub1, b_ekm0, b_ekm1, b_s0, b_s1,
             b_kiix, b_ixki, b_kelog,
             g_c0, g_c1, g_lkm0, g_lkm1, g_lki, g_cki, b_out, sem):
    wid = lax.axis_index("s") * NC + lax.axis_index("c")
    base = wid * NPW

    @pl.loop(0, NCHUNK)
    def _chunk(ch):
        cb = base + ch * C
        rows = pl.ds(cb, C)

        cps = [
            pltpu.async_copy(sub0_h.at[rows], b_sub0, sem),
            pltpu.async_copy(sub1_h.at[rows], b_sub1, sem),
            pltpu.async_copy(ekm0_h.at[rows], b_ekm0, sem),
            pltpu.async_copy(ekm1_h.at[rows], b_ekm1, sem),
            pltpu.async_copy(kiix_h.at[rows], b_kiix, sem),
            pltpu.async_copy(ixki_h.at[rows], b_ixki, sem),
            pltpu.async_copy(s0_h.at[rows], b_s0, sem),
            pltpu.async_copy(s1_h.at[rows], b_s1, sem),
            pltpu.async_copy(kelog_h.at[rows], b_kelog, sem),
        ]
        for cp in cps:
            cp.wait()

        gps = [
            pltpu.async_copy(conc_h.at[b_sub0], g_c0, sem),
            pltpu.async_copy(conc_h.at[b_sub1], g_c1, sem),
            pltpu.async_copy(lkm_h.at[b_ekm0], g_lkm0, sem),
            pltpu.async_copy(lkm_h.at[b_ekm1], g_lkm1, sem),
            pltpu.async_copy(lki_h.at[b_kiix], g_lki, sem),
            pltpu.async_copy(conc_h.at[b_ixki], g_cki, sem),
        ]
        for cp in gps:
            cp.wait()

        @pl.loop(0, G)
        def _compute(t):
            sl = pl.ds(t * L, L)
            r0 = g_c0[sl] * jnp.exp(-g_lkm0[sl])
            r1 = g_c1[sl] * jnp.exp(-g_lkm1[sl])
            main = jnp.exp(b_s0[sl] * _ln1p_pos(r0)
                           + b_s1[sl] * _ln1p_pos(r1))
            denom = main + g_cki[sl] * jnp.exp(-g_lki[sl])
            b_out[sl] = jnp.exp(b_kelog[sl]) * r0 * r1 / denom

        pltpu.sync_copy(b_out, out_h.at[rows])


@jax.jit
def _mm_flux(conc, log_kcat, log_enzyme, log_km, log_ki, stoich, kmix,
             kiix, ixsub, ixki, kmpos, rpos):
    sub0, sub1, ekm0, ekm1, s0, s1, kiix1, ixki1, kelog = _prep(
        kmix, kmpos, rpos, stoich, ixsub, kiix, ixki, log_kcat, log_enzyme)

    mesh = plsc.VectorSubcoreMesh(core_axis_name="c", subcore_axis_name="s")
    cp = pltpu.CompilerParams()
    if "needs_layout_passes" in pltpu.CompilerParams.__dataclass_fields__:
        cp = dataclasses.replace(cp, needs_layout_passes=False)
    f = pl.kernel(
        _mm_body,
        compiler_params=cp,
        out_type=jax.ShapeDtypeStruct((R,), jnp.float32),
        mesh=mesh,
        scratch_types=[
            pltpu.VMEM((C,), jnp.int32),
            pltpu.VMEM((C,), jnp.int32),
            pltpu.VMEM((C,), jnp.int32),
            pltpu.VMEM((C,), jnp.int32),
            pltpu.VMEM((C,), jnp.float32),
            pltpu.VMEM((C,), jnp.float32),
            pltpu.VMEM((C,), jnp.int32),
            pltpu.VMEM((C,), jnp.int32),
            pltpu.VMEM((C,), jnp.float32),
            pltpu.VMEM((C,), jnp.float32),
            pltpu.VMEM((C,), jnp.float32),
            pltpu.VMEM((C,), jnp.float32),
            pltpu.VMEM((C,), jnp.float32),
            pltpu.VMEM((C,), jnp.float32),
            pltpu.VMEM((C,), jnp.float32),
            pltpu.VMEM((C,), jnp.float32),
            pltpu.SemaphoreType.DMA,
        ],
    )
    return f(conc, log_km, log_ki, kelog, sub0, sub1, ekm0, ekm1,
             s0, s1, kiix1, ixki1)


def kernel(conc, log_kcat, log_enzyme, log_km, log_ki,
           reactant_stoichiometry, kcat_ix, enzyme_ix, km_ix, ki_ix,
           ix_substrate, ix_ki_species, substrate_km_positions,
           substrate_reactant_positions):
    del kcat_ix, enzyme_ix
    return _mm_flux(
        conc, log_kcat, log_enzyme, log_km, log_ki,
        reactant_stoichiometry, km_ix, ki_ix, ix_substrate, ix_ki_species,
        substrate_km_positions, substrate_reactant_positions,
    )


# speedup vs baseline: 1.3606x; 1.3606x over previous
import dataclasses
import functools

import jax
import jax.numpy as jnp
from jax import lax
from jax.experimental import pallas as pl
from jax.experimental.pallas import tpu as pltpu
from jax.experimental.pallas import tpu_sc as plsc

R = 262144
NC = 2
NS = 16
NW = NC * NS
NPW = R // NW
BIG = 2048
NBIG = NPW // BIG
MICRO = 32
MPB = BIG // MICRO
L = 16
G = BIG // L

_LN2 = 0.6931471805599453
_SQRT2 = 1.4142135623730951


def _ln1p_pos(r):
    x = r + 1.0
    xi = lax.bitcast_convert_type(x, jnp.int32)
    e = lax.shift_right_logical(xi, 23) - 127
    m = lax.bitcast_convert_type(
        jnp.bitwise_or(jnp.bitwise_and(xi, 0x007FFFFF), 0x3F800000),
        jnp.float32)
    big = m > _SQRT2
    m = jnp.where(big, m * 0.5, m)
    e = e + jnp.where(big, 1, 0)
    u = (m - 1.0) / (m + 1.0)
    u2 = u * u
    p = u * (2.0 + u2 * (0.6666666666666666
                         + u2 * (0.4 + u2 * 0.2857142857142857)))
    return e.astype(jnp.float32) * _LN2 + p


def _mm_body(conc_h, lkcat_h, lenz_h, lkm_h, lki_h, stoich_h, kmix_h, kiix_h,
             ixsub_h, ixki_h, kmpos_h, rpos_h, out_h,
             idx2, rb_st, rb_km, rb_ki, rb_sub, rb_xk, rb_p, rb_q,
             c_sub0, c_sub1, c_ekm0, c_ekm1, c_kiix, c_ixki, c_s0, c_s1,
             g_c0, g_c1, g_lkm0, g_lkm1, g_lki, g_cki,
             b_lkcat, b_lenz, b_out, sem, sem2):
    wid = lax.axis_index("s") * NC + lax.axis_index("c")
    base = wid * NPW
    lane = lax.iota(jnp.int32, L)
    zero = jnp.zeros((L,), jnp.int32)
    one = jnp.ones((L,), jnp.int32)

    @pl.loop(0, NBIG)
    def _big(big):
        bb = base + big * BIG
        rows = pl.ds(bb, BIG)

        lc0 = pltpu.async_copy(lkcat_h.at[rows], b_lkcat, sem2)
        lc1 = pltpu.async_copy(lenz_h.at[rows], b_lenz, sem2)

        @pl.loop(0, MPB)
        def _micro(m):
            mb = bb + m * MICRO
            mrows = pl.ds(mb, MICRO)
            gps = [
                pltpu.async_copy(stoich_h.at[mrows, :], rb_st, sem),
                pltpu.async_copy(kmix_h.at[mrows, :], rb_km, sem),
                pltpu.async_copy(kiix_h.at[mrows, :], rb_ki, sem),
                pltpu.async_copy(ixsub_h.at[mrows, :], rb_sub, sem),
                pltpu.async_copy(ixki_h.at[mrows, :], rb_xk, sem),
                pltpu.async_copy(kmpos_h.at[mrows, :], rb_p, sem),
                pltpu.async_copy(rpos_h.at[mrows, :], rb_q, sem),
            ]
            for cp in gps:
                cp.wait()
            for g in range(2):
                r16 = g * L + lane
                p0 = plsc.load_gather(rb_p, [r16, zero])
                p1 = plsc.load_gather(rb_p, [r16, one])
                q0 = plsc.load_gather(rb_q, [r16, zero])
                q1 = plsc.load_gather(rb_q, [r16, one])
                km0 = plsc.load_gather(rb_km, [r16, zero])
                km1 = plsc.load_gather(rb_km, [r16, one])
                st0 = plsc.load_gather(rb_st, [r16, zero])
                st1 = plsc.load_gather(rb_st, [r16, one])
                sl = pl.ds(m * MICRO + g * L, L)
                c_sub0[sl] = plsc.load_gather(rb_sub, [r16, zero])
                c_sub1[sl] = plsc.load_gather(rb_sub, [r16, one])
                c_kiix[sl] = plsc.load_gather(rb_ki, [r16, zero])
                c_ixki[sl] = plsc.load_gather(rb_xk, [r16, zero])
                c_ekm0[sl] = jnp.where(p0 == 0, km0, km1)
                c_ekm1[sl] = jnp.where(p1 == 0, km0, km1)
                c_s0[sl] = jnp.abs(jnp.where(q0 == 0, st0, st1))
                c_s1[sl] = jnp.abs(jnp.where(q1 == 0, st0, st1))

        lc0.wait()
        lc1.wait()
        gps = [
            pltpu.async_copy(conc_h.at[c_sub0], g_c0, sem2),
            pltpu.async_copy(conc_h.at[c_sub1], g_c1, sem2),
            pltpu.async_copy(lkm_h.at[c_ekm0], g_lkm0, sem2),
            pltpu.async_copy(lkm_h.at[c_ekm1], g_lkm1, sem2),
            pltpu.async_copy(lki_h.at[c_kiix], g_lki, sem2),
            pltpu.async_copy(conc_h.at[c_ixki], g_cki, sem2),
        ]
        for cp in gps:
            cp.wait()

        @pl.loop(0, G)
        def _compute(t):
            sl = pl.ds(t * L, L)
            r0 = g_c0[sl] * jnp.exp(-g_lkm0[sl])
            r1 = g_c1[sl] * jnp.exp(-g_lkm1[sl])
            main = jnp.exp(c_s0[sl] * _ln1p_pos(r0)
                           + c_s1[sl] * _ln1p_pos(r1))
            denom = main + g_cki[sl] * jnp.exp(-g_lki[sl])
            ke = jnp.exp(b_lkcat[sl] + b_lenz[sl])
            b_out[sl] = ke * r0 * r1 / denom

        pltpu.sync_copy(b_out, out_h.at[rows])


@jax.jit
def _mm_flux(conc, log_kcat, log_enzyme, log_km, log_ki, stoich, kmix,
             kiix, ixsub, ixki, kmpos, rpos):
    mesh = plsc.VectorSubcoreMesh(core_axis_name="c", subcore_axis_name="s")
    cp = pltpu.CompilerParams()
    if "needs_layout_passes" in pltpu.CompilerParams.__dataclass_fields__:
        cp = dataclasses.replace(cp, needs_layout_passes=False)
    f = pl.kernel(
        _mm_body,
        compiler_params=cp,
        out_type=jax.ShapeDtypeStruct((R,), jnp.float32),
        mesh=mesh,
        scratch_types=[
            pltpu.VMEM((MICRO,), jnp.int32),
            pltpu.VMEM((MICRO, 2), jnp.float32),
            pltpu.VMEM((MICRO, 2), jnp.int32),
            pltpu.VMEM((MICRO, 1), jnp.int32),
            pltpu.VMEM((MICRO, 2), jnp.int32),
            pltpu.VMEM((MICRO, 1), jnp.int32),
            pltpu.VMEM((MICRO, 2), jnp.int32),
            pltpu.VMEM((MICRO, 2), jnp.int32),
            pltpu.VMEM((BIG,), jnp.int32),
            pltpu.VMEM((BIG,), jnp.int32),
            pltpu.VMEM((BIG,), jnp.int32),
            pltpu.VMEM((BIG,), jnp.int32),
            pltpu.VMEM((BIG,), jnp.int32),
            pltpu.VMEM((BIG,), jnp.int32),
            pltpu.VMEM((BIG,), jnp.float32),
            pltpu.VMEM((BIG,), jnp.float32),
            pltpu.VMEM((BIG,), jnp.float32),
            pltpu.VMEM((BIG,), jnp.float32),
            pltpu.VMEM((BIG,), jnp.float32),
            pltpu.VMEM((BIG,), jnp.float32),
            pltpu.VMEM((BIG,), jnp.float32),
            pltpu.VMEM((BIG,), jnp.float32),
            pltpu.VMEM((BIG,), jnp.float32),
            pltpu.VMEM((BIG,), jnp.float32),
            pltpu.VMEM((BIG,), jnp.float32),
            pltpu.SemaphoreType.DMA,
            pltpu.SemaphoreType.DMA,
        ],
    )
    return f(conc, log_kcat, log_enzyme, log_km, log_ki, stoich, kmix,
             kiix, ixsub, ixki, kmpos, rpos)


def kernel(conc, log_kcat, log_enzyme, log_km, log_ki,
           reactant_stoichiometry, kcat_ix, enzyme_ix, km_ix, ki_ix,
           ix_substrate, ix_ki_species, substrate_km_positions,
           substrate_reactant_positions):
    del kcat_ix, enzyme_ix
    return _mm_flux(
        conc, log_kcat, log_enzyme, log_km, log_ki,
        reactant_stoichiometry, km_ix, ki_ix, ix_substrate, ix_ki_species,
        substrate_km_positions, substrate_reactant_positions,
    )


# speedup vs baseline: 1.5187x; 1.1162x over previous
import dataclasses
import functools

import jax
import jax.numpy as jnp
from jax import lax
from jax.experimental import pallas as pl
from jax.experimental.pallas import tpu as pltpu
from jax.experimental.pallas import tpu_sc as plsc

R = 262144
NC = 2
NS = 16
NW = NC * NS
NPW = R // NW
BIG = 2048
NBIG = NPW // BIG
MICRO = 32
MPB = BIG // MICRO
L = 16
G = BIG // L

_LN2 = 0.6931471805599453
_SQRT2 = 1.4142135623730951


def _ln1p_pos(r):
    x = r + 1.0
    xi = lax.bitcast_convert_type(x, jnp.int32)
    e = lax.shift_right_logical(xi, 23) - 127
    m = lax.bitcast_convert_type(
        jnp.bitwise_or(jnp.bitwise_and(xi, 0x007FFFFF), 0x3F800000),
        jnp.float32)
    big = m > _SQRT2
    m = jnp.where(big, m * 0.5, m)
    e = e + jnp.where(big, 1, 0)
    u = (m - 1.0) / (m + 1.0)
    u2 = u * u
    p = u * (2.0 + u2 * (0.6666666666666666
                         + u2 * (0.4 + u2 * 0.2857142857142857)))
    return e.astype(jnp.float32) * _LN2 + p


def _mm_body(conc_h, lkcat_h, lenz_h, lkm_h, lki_h, stoich_h, kmix_h, kiix_h,
             ixsub_h, ixki_h, kmpos_h, rpos_h, out_h,
             rb_st0, rb_km0, rb_ki0, rb_sub0, rb_xk0, rb_p0, rb_q0,
             rb_st1, rb_km1, rb_ki1, rb_sub1, rb_xk1, rb_p1, rb_q1,
             c_sub0, c_sub1, c_ekm0, c_ekm1, c_kiix, c_ixki, c_s0, c_s1,
             g_c0, g_c1, g_lkm0, g_lkm1, g_lki, g_cki,
             b_lkcat, b_lenz, b_out, semA, semB, sem2):
    wid = lax.axis_index("s") * NC + lax.axis_index("c")
    base = wid * NPW
    lane = lax.iota(jnp.int32, L)
    zero = jnp.zeros((L,), jnp.int32)
    one = jnp.ones((L,), jnp.int32)

    bufs = [
        (rb_st0, rb_km0, rb_ki0, rb_sub0, rb_xk0, rb_p0, rb_q0, semA),
        (rb_st1, rb_km1, rb_ki1, rb_sub1, rb_xk1, rb_p1, rb_q1, semB),
    ]

    def issue_micro(mb, par):
        rb_st, rb_km, rb_ki, rb_sub, rb_xk, rb_p, rb_q, sem = bufs[par]
        mrows = pl.ds(mb, MICRO)
        pltpu.async_copy(stoich_h.at[mrows, :], rb_st, sem)
        pltpu.async_copy(kmix_h.at[mrows, :], rb_km, sem)
        pltpu.async_copy(kiix_h.at[mrows, :], rb_ki, sem)
        pltpu.async_copy(ixsub_h.at[mrows, :], rb_sub, sem)
        pltpu.async_copy(ixki_h.at[mrows, :], rb_xk, sem)
        pltpu.async_copy(kmpos_h.at[mrows, :], rb_p, sem)
        pltpu.async_copy(rpos_h.at[mrows, :], rb_q, sem)

    def drain_micro(par):
        rb_st, rb_km, rb_ki, rb_sub, rb_xk, rb_p, rb_q, sem = bufs[par]
        m0 = pl.ds(0, MICRO)
        pltpu.make_async_copy(stoich_h.at[m0, :], rb_st, sem).wait()
        pltpu.make_async_copy(kmix_h.at[m0, :], rb_km, sem).wait()
        pltpu.make_async_copy(kiix_h.at[m0, :], rb_ki, sem).wait()
        pltpu.make_async_copy(ixsub_h.at[m0, :], rb_sub, sem).wait()
        pltpu.make_async_copy(ixki_h.at[m0, :], rb_xk, sem).wait()
        pltpu.make_async_copy(kmpos_h.at[m0, :], rb_p, sem).wait()
        pltpu.make_async_copy(rpos_h.at[m0, :], rb_q, sem).wait()

    def compact_micro(m, par):
        rb_st, rb_km, rb_ki, rb_sub, rb_xk, rb_p, rb_q, _ = bufs[par]
        for g in range(2):
            r16 = g * L + lane
            p0 = plsc.load_gather(rb_p, [r16, zero])
            p1 = plsc.load_gather(rb_p, [r16, one])
            q0 = plsc.load_gather(rb_q, [r16, zero])
            q1 = plsc.load_gather(rb_q, [r16, one])
            km0 = plsc.load_gather(rb_km, [r16, zero])
            km1 = plsc.load_gather(rb_km, [r16, one])
            st0 = plsc.load_gather(rb_st, [r16, zero])
            st1 = plsc.load_gather(rb_st, [r16, one])
            sl = pl.ds(m * MICRO + g * L, L)
            c_sub0[sl] = plsc.load_gather(rb_sub, [r16, zero])
            c_sub1[sl] = plsc.load_gather(rb_sub, [r16, one])
            c_kiix[sl] = plsc.load_gather(rb_ki, [r16, zero])
            c_ixki[sl] = plsc.load_gather(rb_xk, [r16, zero])
            c_ekm0[sl] = jnp.where(p0 == 0, km0, km1)
            c_ekm1[sl] = jnp.where(p1 == 0, km0, km1)
            c_s0[sl] = jnp.abs(jnp.where(q0 == 0, st0, st1))
            c_s1[sl] = jnp.abs(jnp.where(q1 == 0, st0, st1))

    @pl.loop(0, NBIG)
    def _big(big):
        bb = base + big * BIG
        rows = pl.ds(bb, BIG)

        lc0 = pltpu.async_copy(lkcat_h.at[rows], b_lkcat, sem2)
        lc1 = pltpu.async_copy(lenz_h.at[rows], b_lenz, sem2)

        issue_micro(bb, 0)

        @pl.loop(0, MPB // 2)
        def _micro(mm):
            m0 = mm * 2
            issue_micro(bb + (m0 + 1) * MICRO, 1)
            drain_micro(0)
            compact_micro(m0, 0)

            @pl.when(m0 + 2 < MPB)
            def _():
                issue_micro(bb + (m0 + 2) * MICRO, 0)

            drain_micro(1)
            compact_micro(m0 + 1, 1)

        lc0.wait()
        lc1.wait()
        gps = [
            pltpu.async_copy(conc_h.at[c_sub0], g_c0, sem2),
            pltpu.async_copy(conc_h.at[c_sub1], g_c1, sem2),
            pltpu.async_copy(lkm_h.at[c_ekm0], g_lkm0, sem2),
            pltpu.async_copy(lkm_h.at[c_ekm1], g_lkm1, sem2),
            pltpu.async_copy(lki_h.at[c_kiix], g_lki, sem2),
            pltpu.async_copy(conc_h.at[c_ixki], g_cki, sem2),
        ]
        for cp in gps:
            cp.wait()

        @pl.loop(0, G)
        def _compute(t):
            sl = pl.ds(t * L, L)
            r0 = g_c0[sl] * jnp.exp(-g_lkm0[sl])
            r1 = g_c1[sl] * jnp.exp(-g_lkm1[sl])
            main = jnp.exp(c_s0[sl] * _ln1p_pos(r0)
                           + c_s1[sl] * _ln1p_pos(r1))
            denom = main + g_cki[sl] * jnp.exp(-g_lki[sl])
            ke = jnp.exp(b_lkcat[sl] + b_lenz[sl])
            b_out[sl] = ke * r0 * r1 / denom

        pltpu.sync_copy(b_out, out_h.at[rows])


@jax.jit
def _mm_flux(conc, log_kcat, log_enzyme, log_km, log_ki, stoich, kmix,
             kiix, ixsub, ixki, kmpos, rpos):
    mesh = plsc.VectorSubcoreMesh(core_axis_name="c", subcore_axis_name="s")
    cp = pltpu.CompilerParams()
    if "needs_layout_passes" in pltpu.CompilerParams.__dataclass_fields__:
        cp = dataclasses.replace(cp, needs_layout_passes=False)
    f = pl.kernel(
        _mm_body,
        compiler_params=cp,
        out_type=jax.ShapeDtypeStruct((R,), jnp.float32),
        mesh=mesh,
        scratch_types=[
            pltpu.VMEM((MICRO, 2), jnp.float32),
            pltpu.VMEM((MICRO, 2), jnp.int32),
            pltpu.VMEM((MICRO, 1), jnp.int32),
            pltpu.VMEM((MICRO, 2), jnp.int32),
            pltpu.VMEM((MICRO, 1), jnp.int32),
            pltpu.VMEM((MICRO, 2), jnp.int32),
            pltpu.VMEM((MICRO, 2), jnp.int32),
            pltpu.VMEM((MICRO, 2), jnp.float32),
            pltpu.VMEM((MICRO, 2), jnp.int32),
            pltpu.VMEM((MICRO, 1), jnp.int32),
            pltpu.VMEM((MICRO, 2), jnp.int32),
            pltpu.VMEM((MICRO, 1), jnp.int32),
            pltpu.VMEM((MICRO, 2), jnp.int32),
            pltpu.VMEM((MICRO, 2), jnp.int32),
            pltpu.VMEM((BIG,), jnp.int32),
            pltpu.VMEM((BIG,), jnp.int32),
            pltpu.VMEM((BIG,), jnp.int32),
            pltpu.VMEM((BIG,), jnp.int32),
            pltpu.VMEM((BIG,), jnp.int32),
            pltpu.VMEM((BIG,), jnp.int32),
            pltpu.VMEM((BIG,), jnp.float32),
            pltpu.VMEM((BIG,), jnp.float32),
            pltpu.VMEM((BIG,), jnp.float32),
            pltpu.VMEM((BIG,), jnp.float32),
            pltpu.VMEM((BIG,), jnp.float32),
            pltpu.VMEM((BIG,), jnp.float32),
            pltpu.VMEM((BIG,), jnp.float32),
            pltpu.VMEM((BIG,), jnp.float32),
            pltpu.VMEM((BIG,), jnp.float32),
            pltpu.VMEM((BIG,), jnp.float32),
            pltpu.VMEM((BIG,), jnp.float32),
            pltpu.SemaphoreType.DMA,
            pltpu.SemaphoreType.DMA,
            pltpu.SemaphoreType.DMA,
        ],
    )
    return f(conc, log_kcat, log_enzyme, log_km, log_ki, stoich, kmix,
             kiix, ixsub, ixki, kmpos, rpos)


def kernel(conc, log_kcat, log_enzyme, log_km, log_ki,
           reactant_stoichiometry, kcat_ix, enzyme_ix, km_ix, ki_ix,
           ix_substrate, ix_ki_species, substrate_km_positions,
           substrate_reactant_positions):
    del kcat_ix, enzyme_ix
    return _mm_flux(
        conc, log_kcat, log_enzyme, log_km, log_ki,
        reactant_stoichiometry, km_ix, ki_ix, ix_substrate, ix_ki_species,
        substrate_km_positions, substrate_reactant_positions,
    )


# speedup vs baseline: 2.0539x; 1.3524x over previous
import dataclasses
import functools

import jax
import jax.numpy as jnp
from jax import lax
from jax.experimental import pallas as pl
from jax.experimental.pallas import tpu as pltpu
from jax.experimental.pallas import tpu_sc as plsc

R = 262144
NC = 2
NS = 16
NW = NC * NS
NPW = R // NW
BIG = 2048
NBIG = NPW // BIG
MICRO = 64
MPB = BIG // MICRO
L = 16
G = BIG // L

_LN2 = 0.6931471805599453
_SQRT2 = 1.4142135623730951


def _ln1p_pos(r):
    x = r + 1.0
    xi = lax.bitcast_convert_type(x, jnp.int32)
    e = lax.shift_right_logical(xi, 23) - 127
    m = lax.bitcast_convert_type(
        jnp.bitwise_or(jnp.bitwise_and(xi, 0x007FFFFF), 0x3F800000),
        jnp.float32)
    big = m > _SQRT2
    m = jnp.where(big, m * 0.5, m)
    e = e + jnp.where(big, 1, 0)
    u = (m - 1.0) / (m + 1.0)
    u2 = u * u
    p = u * (2.0 + u2 * (0.6666666666666666
                         + u2 * (0.4 + u2 * 0.2857142857142857)))
    return e.astype(jnp.float32) * _LN2 + p


def _mm_body(conc_h, lkcat_h, lenz_h, lkm_h, lki_h, stoich_h, kmix_h, kiix_h,
             ixsub_h, ixki_h, kmpos_h, rpos_h, out_h,
             rb_st0, rb_km0, rb_sub0, rb_p0, rb_q0,
             rb_st1, rb_km1, rb_sub1, rb_p1, rb_q1,
             c_sub0, c_sub1, c_ekm0, c_ekm1, c_kiix, c_ixki, c_s0, c_s1,
             g_c0, g_c1, g_lkm0, g_lkm1, g_lki, g_cki,
             b_lkcat, b_lenz, b_out, semA, semB, sem2):
    wid = lax.axis_index("s") * NC + lax.axis_index("c")
    base = wid * NPW
    lane = lax.iota(jnp.int32, L)
    zero = jnp.zeros((L,), jnp.int32)
    one = jnp.ones((L,), jnp.int32)

    bufs = [
        (rb_st0, rb_km0, rb_sub0, rb_p0, rb_q0, semA),
        (rb_st1, rb_km1, rb_sub1, rb_p1, rb_q1, semB),
    ]

    def issue_micro(mb, par):
        rb_st, rb_km, rb_sub, rb_p, rb_q, sem = bufs[par]
        mrows = pl.ds(mb, MICRO)
        pltpu.async_copy(stoich_h.at[mrows, :], rb_st, sem)
        pltpu.async_copy(kmix_h.at[mrows, :], rb_km, sem)
        pltpu.async_copy(ixsub_h.at[mrows, :], rb_sub, sem)
        pltpu.async_copy(kmpos_h.at[mrows, :], rb_p, sem)
        pltpu.async_copy(rpos_h.at[mrows, :], rb_q, sem)

    def drain_micro(par):
        rb_st, rb_km, rb_sub, rb_p, rb_q, sem = bufs[par]
        m0 = pl.ds(0, MICRO)
        pltpu.make_async_copy(stoich_h.at[m0, :], rb_st, sem).wait()
        pltpu.make_async_copy(kmix_h.at[m0, :], rb_km, sem).wait()
        pltpu.make_async_copy(ixsub_h.at[m0, :], rb_sub, sem).wait()
        pltpu.make_async_copy(kmpos_h.at[m0, :], rb_p, sem).wait()
        pltpu.make_async_copy(rpos_h.at[m0, :], rb_q, sem).wait()

    def compact_micro(m, par):
        rb_st, rb_km, rb_sub, rb_p, rb_q, _ = bufs[par]
        for g in range(MICRO // L):
            r16 = g * L + lane
            p0 = plsc.load_gather(rb_p, [r16, zero])
            p1 = plsc.load_gather(rb_p, [r16, one])
            q0 = plsc.load_gather(rb_q, [r16, zero])
            q1 = plsc.load_gather(rb_q, [r16, one])
            km0 = plsc.load_gather(rb_km, [r16, zero])
            km1 = plsc.load_gather(rb_km, [r16, one])
            st0 = plsc.load_gather(rb_st, [r16, zero])
            st1 = plsc.load_gather(rb_st, [r16, one])
            sl = pl.ds(m * MICRO + g * L, L)
            c_sub0[sl] = plsc.load_gather(rb_sub, [r16, zero])
            c_sub1[sl] = plsc.load_gather(rb_sub, [r16, one])
            c_ekm0[sl] = jnp.where(p0 == 0, km0, km1)
            c_ekm1[sl] = jnp.where(p1 == 0, km0, km1)
            c_s0[sl] = jnp.abs(jnp.where(q0 == 0, st0, st1))
            c_s1[sl] = jnp.abs(jnp.where(q1 == 0, st0, st1))

    @pl.loop(0, NBIG)
    def _big(big):
        bb = base + big * BIG
        rows = pl.ds(bb, BIG)

        lc0 = pltpu.async_copy(lkcat_h.at[rows], b_lkcat, sem2)
        lc1 = pltpu.async_copy(lenz_h.at[rows], b_lenz, sem2)
        lc2 = pltpu.async_copy(kiix_h.at[rows], c_kiix, sem2)
        lc3 = pltpu.async_copy(ixki_h.at[rows], c_ixki, sem2)

        issue_micro(bb, 0)

        @pl.loop(0, MPB // 2)
        def _micro(mm):
            m0 = mm * 2
            issue_micro(bb + (m0 + 1) * MICRO, 1)
            drain_micro(0)
            compact_micro(m0, 0)

            @pl.when(m0 + 2 < MPB)
            def _():
                issue_micro(bb + (m0 + 2) * MICRO, 0)

            drain_micro(1)
            compact_micro(m0 + 1, 1)

        lc0.wait()
        lc1.wait()
        lc2.wait()
        lc3.wait()
        gps = [
            pltpu.async_copy(conc_h.at[c_sub0], g_c0, sem2),
            pltpu.async_copy(conc_h.at[c_sub1], g_c1, sem2),
            pltpu.async_copy(lkm_h.at[c_ekm0], g_lkm0, sem2),
            pltpu.async_copy(lkm_h.at[c_ekm1], g_lkm1, sem2),
            pltpu.async_copy(lki_h.at[c_kiix], g_lki, sem2),
            pltpu.async_copy(conc_h.at[c_ixki], g_cki, sem2),
        ]
        for cp in gps:
            cp.wait()

        @pl.loop(0, G)
        def _compute(t):
            sl = pl.ds(t * L, L)
            r0 = g_c0[sl] * jnp.exp(-g_lkm0[sl])
            r1 = g_c1[sl] * jnp.exp(-g_lkm1[sl])
            main = jnp.exp(c_s0[sl] * _ln1p_pos(r0)
                           + c_s1[sl] * _ln1p_pos(r1))
            denom = main + g_cki[sl] * jnp.exp(-g_lki[sl])
            ke = jnp.exp(b_lkcat[sl] + b_lenz[sl])
            b_out[sl] = ke * r0 * r1 / denom

        pltpu.sync_copy(b_out, out_h.at[rows])


@jax.jit
def _mm_flux(conc, log_kcat, log_enzyme, log_km, log_ki, stoich, kmix,
             kiix, ixsub, ixki, kmpos, rpos):
    mesh = plsc.VectorSubcoreMesh(core_axis_name="c", subcore_axis_name="s")
    cp = pltpu.CompilerParams()
    if "needs_layout_passes" in pltpu.CompilerParams.__dataclass_fields__:
        cp = dataclasses.replace(cp, needs_layout_passes=False)
    f = pl.kernel(
        _mm_body,
        compiler_params=cp,
        out_type=jax.ShapeDtypeStruct((R,), jnp.float32),
        mesh=mesh,
        scratch_types=[
            pltpu.VMEM((MICRO, 2), jnp.float32),
            pltpu.VMEM((MICRO, 2), jnp.int32),
            pltpu.VMEM((MICRO, 2), jnp.int32),
            pltpu.VMEM((MICRO, 2), jnp.int32),
            pltpu.VMEM((MICRO, 2), jnp.int32),
            pltpu.VMEM((MICRO, 2), jnp.float32),
            pltpu.VMEM((MICRO, 2), jnp.int32),
            pltpu.VMEM((MICRO, 2), jnp.int32),
            pltpu.VMEM((MICRO, 2), jnp.int32),
            pltpu.VMEM((MICRO, 2), jnp.int32),
            pltpu.VMEM((BIG,), jnp.int32),
            pltpu.VMEM((BIG,), jnp.int32),
            pltpu.VMEM((BIG,), jnp.int32),
            pltpu.VMEM((BIG,), jnp.int32),
            pltpu.VMEM((BIG,), jnp.int32),
            pltpu.VMEM((BIG,), jnp.int32),
            pltpu.VMEM((BIG,), jnp.float32),
            pltpu.VMEM((BIG,), jnp.float32),
            pltpu.VMEM((BIG,), jnp.float32),
            pltpu.VMEM((BIG,), jnp.float32),
            pltpu.VMEM((BIG,), jnp.float32),
            pltpu.VMEM((BIG,), jnp.float32),
            pltpu.VMEM((BIG,), jnp.float32),
            pltpu.VMEM((BIG,), jnp.float32),
            pltpu.VMEM((BIG,), jnp.float32),
            pltpu.VMEM((BIG,), jnp.float32),
            pltpu.VMEM((BIG,), jnp.float32),
            pltpu.SemaphoreType.DMA,
            pltpu.SemaphoreType.DMA,
            pltpu.SemaphoreType.DMA,
        ],
    )
    return f(conc, log_kcat, log_enzyme, log_km, log_ki, stoich, kmix,
             kiix, ixsub, ixki, kmpos, rpos)


def kernel(conc, log_kcat, log_enzyme, log_km, log_ki,
           reactant_stoichiometry, kcat_ix, enzyme_ix, km_ix, ki_ix,
           ix_substrate, ix_ki_species, substrate_km_positions,
           substrate_reactant_positions):
    del kcat_ix, enzyme_ix
    return _mm_flux(
        conc, log_kcat, log_enzyme, log_km, log_ki,
        reactant_stoichiometry, km_ix, ki_ix.reshape(-1), ix_substrate,
        ix_ki_species.reshape(-1),
        substrate_km_positions, substrate_reactant_positions,
    )
